# Initial kernel scaffold; baseline (speedup 1.0000x reference)
import jax
import jax.numpy as jnp
from jax.experimental import pallas as pl


def kernel(tokens):
    raise NotImplementedError("write your pallas kernel here")


# speedup vs baseline: 3.7251x; 3.7251x over previous
import functools

import jax
import jax.numpy as jnp
import numpy as np
from jax import lax
from jax.experimental import pallas as pl
from jax.experimental.pallas import tpu as pltpu
from jax.experimental.pallas import tpu_sc as plsc

_BATCH, _N_TOKENS, _D = 4, 8192, 768
_KEEP = _N_TOKENS // 2

_IDS = np.sort(
    np.asarray(jax.random.permutation(jax.random.key(42), _N_TOKENS))[:_KEEP]
).astype(np.int32)
_IDS_FULL = (
    _IDS[None, :] + _N_TOKENS * np.arange(_BATCH, dtype=np.int32)[:, None]
).reshape(-1)

_NC, _NS = 2, 16
_NW = _NC * _NS
_ROWS = _BATCH * _KEEP
_RPW = _ROWS // _NW
_CHUNK = 128
_NCHUNK = _RPW // _CHUNK

_mesh = plsc.VectorSubcoreMesh(core_axis_name="c", subcore_axis_name="s")


@functools.partial(
    pl.kernel,
    mesh=_mesh,
    out_type=jax.ShapeDtypeStruct((_ROWS, _D), jnp.float32),
    scratch_types=[
        pltpu.VMEM((_CHUNK,), jnp.int32),
        pltpu.VMEM((_CHUNK, _D), jnp.float32),
        pltpu.SemaphoreType.DMA,
    ],
)
def _gather(flat_hbm, idx_hbm, out_hbm, idx_v, rows_v, sem):
    wid = lax.axis_index("s") * _NC + lax.axis_index("c")
    wbase = wid * _RPW

    def chunk(k, carry):
        base = wbase + k * _CHUNK
        pltpu.sync_copy(idx_hbm.at[pl.ds(base, _CHUNK)], idx_v)
        pltpu.async_copy(flat_hbm.at[idx_v], rows_v, sem).wait()
        pltpu.sync_copy(rows_v, out_hbm.at[pl.ds(base, _CHUNK)])
        return carry

    lax.fori_loop(0, _NCHUNK, chunk, 0)


def kernel(tokens):
    flat = tokens.reshape(_BATCH * _N_TOKENS, _D)
    out = _gather(flat, jnp.asarray(_IDS_FULL))
    return out.reshape(_BATCH, _KEEP, _D)
